# baseline (speedup 1.0000x reference)
import functools

import jax
import jax.numpy as jnp
from jax import lax
from jax.experimental import pallas as pl
from jax.experimental.pallas import tpu as pltpu
from jax.experimental.pallas import tpu_sc as plsc

_NC = 2
_NS = 16
_NW = _NC * _NS
_CHUNK = 128


def _make_segment_sum(n, d, e_pad, agg_rows):
    per_w = e_pad // _NW
    n_chunks = per_w // _CHUNK
    rows_per_tile = agg_rows // _NS
    mesh = plsc.VectorSubcoreMesh(core_axis_name="c", subcore_axis_name="s")

    def body(h_hbm, src_hbm, dst_hbm, zero_hbm, out_hbm,
             agg_sh, src0, src1, dst0, dst1, rows0, rows1, sem0, sem1):
        c = lax.axis_index("c")
        s = lax.axis_index("s")
        wid = s * _NC + c
        src_v = (src0, src1)
        dst_v = (dst0, dst1)
        rows_v = (rows0, rows1)
        sem = (sem0, sem1)
        pltpu.sync_copy(zero_hbm, agg_sh.at[pl.ds(s * rows_per_tile, rows_per_tile)])
        plsc.subcore_barrier()

        base = wid * per_w

        def chunk(k, carry):
            off = base + k * _CHUNK
            pltpu.sync_copy(src_hbm.at[pl.ds(off, _CHUNK)], src_v[0])
            pltpu.sync_copy(dst_hbm.at[pl.ds(off, _CHUNK)], dst_v[0])
            pltpu.async_copy(h_hbm.at[src_v[0]], rows_v[0], sem[0]).wait()
            pltpu.sync_copy(rows_v[0], agg_sh.at[dst_v[0]], add=True)
            return carry

        lax.fori_loop(0, n_chunks, chunk, 0)
        plsc.subcore_barrier()
        pltpu.sync_copy(agg_sh.at[pl.ds(s * rows_per_tile, rows_per_tile)],
                        out_hbm.at[c, pl.ds(s * rows_per_tile, rows_per_tile)])

    return pl.kernel(
        body,
        out_type=jax.ShapeDtypeStruct((_NC, agg_rows, d), jnp.float32),
        mesh=mesh,
        scratch_types=[
            pltpu.VMEM_SHARED((agg_rows, d), jnp.float32),
            pltpu.VMEM((_CHUNK,), jnp.int32),
            pltpu.VMEM((_CHUNK,), jnp.int32),
            pltpu.VMEM((_CHUNK,), jnp.int32),
            pltpu.VMEM((_CHUNK,), jnp.int32),
            pltpu.VMEM((_CHUNK, d), jnp.float32),
            pltpu.VMEM((_CHUNK, d), jnp.float32),
            pltpu.SemaphoreType.DMA,
            pltpu.SemaphoreType.DMA,
        ],
    )


def _bn(x, g, b):
    mu = jnp.mean(x, axis=0, keepdims=True)
    var = jnp.mean((x - mu) ** 2, axis=0, keepdims=True)
    return g * (x - mu) * lax.rsqrt(var + 1e-5) + b


def _dense_body(scale_ref, h_ref, p0_ref, p1_ref, w1_ref, b1_ref, g1_ref,
                be1_ref, w2_ref, b2_ref, g2_ref, be2_ref, bng_ref, bnb_ref,
                out_ref, *, n, final_relu):
    agg = p0_ref[0:n, :] + p1_ref[0:n, :]
    h2 = h_ref[...] * scale_ref[0] + agg
    a = jnp.dot(h2, w1_ref[...], preferred_element_type=jnp.float32) + b1_ref[...]
    a = jnp.maximum(_bn(a, g1_ref[...], be1_ref[...]), 0.0)
    a = jnp.dot(a, w2_ref[...], preferred_element_type=jnp.float32) + b2_ref[...]
    a = jnp.maximum(_bn(a, g2_ref[...], be2_ref[...]), 0.0)
    a = _bn(a, bng_ref[...], bnb_ref[...])
    if final_relu:
        a = jnp.maximum(a, 0.0)
    out_ref[...] = a


def _make_dense(n, d, final_relu):
    vmem = pl.BlockSpec(memory_space=pltpu.VMEM)
    return pl.pallas_call(
        functools.partial(_dense_body, n=n, final_relu=final_relu),
        out_shape=jax.ShapeDtypeStruct((n, d), jnp.float32),
        in_specs=[pl.BlockSpec(memory_space=pltpu.SMEM)] + [vmem] * 13,
        out_specs=vmem,
    )


def kernel(x, edge_index, eps, W1, b1, g1, be1, W2, b2, g2, be2, bn_g, bn_b):
    n, d = x.shape
    e = edge_index.shape[1]
    num_layers = W1.shape[0]

    agg_rows = ((n + _NS - 1) // _NS + 7) // 8 * 8 * _NS
    junk_row = n
    quantum = 2 * _NW * _CHUNK
    e_pad = ((e + quantum - 1) // quantum) * quantum

    src = edge_index[0]
    dst = edge_index[1]
    pad = e_pad + _CHUNK - e
    src_p = jnp.concatenate([src, jnp.zeros((pad,), jnp.int32)])
    dst_p = jnp.concatenate([dst, jnp.full((pad,), junk_row, jnp.int32)])
    zero_block = jnp.zeros((agg_rows // _NS, d), jnp.float32)

    seg_sum = _make_segment_sum(n, d, e_pad, agg_rows)

    h = x
    for i in range(num_layers):
        parts = seg_sum(h, src_p, dst_p, zero_block)
        scale = (1.0 + eps[i]).reshape(1)
        dense = _make_dense(n, d, final_relu=(i < num_layers - 1))
        h = dense(scale, h, parts[0], parts[1],
                  W1[i], b1[i].reshape(1, d), g1[i].reshape(1, d),
                  be1[i].reshape(1, d), W2[i], b2[i].reshape(1, d),
                  g2[i].reshape(1, d), be2[i].reshape(1, d),
                  bn_g[i].reshape(1, d), bn_b[i].reshape(1, d))
    return h


# speedup vs baseline: 2.1075x; 2.1075x over previous
import functools

import jax
import jax.numpy as jnp
from jax import lax
from jax.experimental import pallas as pl
from jax.experimental.pallas import tpu as pltpu
from jax.experimental.pallas import tpu_sc as plsc

_NC = 2
_NS = 16
_NW = _NC * _NS
_CHUNK = 128


def _make_segment_sum(n, d, e_pad, agg_rows):
    per_w = e_pad // _NW
    n_chunks = per_w // _CHUNK
    rows_per_tile = agg_rows // _NS
    mesh = plsc.VectorSubcoreMesh(core_axis_name="c", subcore_axis_name="s")

    def body(h_hbm, src_hbm, dst_hbm, zero_hbm, out_hbm,
             agg_sh, src0, src1, dst0, dst1, rows0, rows1, sem0, sem1):
        c = lax.axis_index("c")
        s = lax.axis_index("s")
        wid = s * _NC + c
        src_v = (src0, src1)
        dst_v = (dst0, dst1)
        rows_v = (rows0, rows1)
        sem = (sem0, sem1)
        pltpu.sync_copy(zero_hbm, agg_sh.at[pl.ds(s * rows_per_tile, rows_per_tile)])
        plsc.subcore_barrier()

        base = wid * per_w

        def chunk(k, carry):
            off = base + k * _CHUNK
            pltpu.sync_copy(src_hbm.at[pl.ds(off, _CHUNK)], src_v[0])
            pltpu.sync_copy(dst_hbm.at[pl.ds(off, _CHUNK)], dst_v[0])
            pltpu.async_copy(h_hbm.at[src_v[0]], rows_v[0], sem[0]).wait()
            pltpu.sync_copy(rows_v[0], agg_sh.at[dst_v[0]], add=True)
            return carry

        lax.fori_loop(0, n_chunks, chunk, 0)
        plsc.subcore_barrier()
        pltpu.sync_copy(agg_sh.at[pl.ds(s * rows_per_tile, rows_per_tile)],
                        out_hbm.at[c, pl.ds(s * rows_per_tile, rows_per_tile)])

    return pl.kernel(
        body,
        out_type=jax.ShapeDtypeStruct((_NC, agg_rows, d), jnp.float32),
        mesh=mesh,
        scratch_types=[
            pltpu.VMEM_SHARED((agg_rows, d), jnp.float32),
            pltpu.VMEM((_CHUNK,), jnp.int32),
            pltpu.VMEM((_CHUNK,), jnp.int32),
            pltpu.VMEM((_CHUNK,), jnp.int32),
            pltpu.VMEM((_CHUNK,), jnp.int32),
            pltpu.VMEM((_CHUNK, d), jnp.float32),
            pltpu.VMEM((_CHUNK, d), jnp.float32),
            pltpu.SemaphoreType.DMA,
            pltpu.SemaphoreType.DMA,
        ],
    )


def _bn(x, g, b):
    mu = jnp.mean(x, axis=0, keepdims=True)
    var = jnp.mean((x - mu) ** 2, axis=0, keepdims=True)
    return g * (x - mu) * lax.rsqrt(var + 1e-5) + b


def _dense_body(scale_ref, h_ref, p0_ref, p1_ref, w1_ref, b1_ref, g1_ref,
                be1_ref, w2_ref, b2_ref, g2_ref, be2_ref, bng_ref, bnb_ref,
                out_ref, *, n, final_relu):
    agg = p0_ref[0:n, :] + p1_ref[0:n, :]
    h2 = h_ref[...] * scale_ref[0] + agg
    a = jnp.dot(h2, w1_ref[...], preferred_element_type=jnp.float32) + b1_ref[...]
    a = jnp.maximum(_bn(a, g1_ref[...], be1_ref[...]), 0.0)
    a = jnp.dot(a, w2_ref[...], preferred_element_type=jnp.float32) + b2_ref[...]
    a = jnp.maximum(_bn(a, g2_ref[...], be2_ref[...]), 0.0)
    a = _bn(a, bng_ref[...], bnb_ref[...])
    if final_relu:
        a = jnp.maximum(a, 0.0)
    out_ref[...] = a


def _make_dense(n, d, final_relu):
    vmem = pl.BlockSpec(memory_space=pltpu.VMEM)
    return pl.pallas_call(
        functools.partial(_dense_body, n=n, final_relu=final_relu),
        out_shape=jax.ShapeDtypeStruct((n, d), jnp.float32),
        in_specs=[pl.BlockSpec(memory_space=pltpu.SMEM)] + [vmem] * 13,
        out_specs=vmem,
    )


def kernel(x, edge_index, eps, W1, b1, g1, be1, W2, b2, g2, be2, bn_g, bn_b):
    n, d = x.shape
    e = edge_index.shape[1]
    num_layers = W1.shape[0]

    agg_rows = ((n + _NS - 1) // _NS + 7) // 8 * 8 * _NS
    junk_row = n
    quantum = 2 * _NW * _CHUNK
    e_pad = ((e + quantum - 1) // quantum) * quantum

    src = edge_index[0]
    dst = edge_index[1]
    per_w_real = e // _NW
    per_w = e_pad // _NW
    pad_w = per_w - per_w_real
    n_junk = agg_rows - n
    pad_src = jnp.broadcast_to((jnp.arange(pad_w, dtype=jnp.int32) * 41) % n,
                               (_NW, pad_w))
    pad_dst = jnp.broadcast_to(junk_row + (jnp.arange(pad_w, dtype=jnp.int32) % n_junk),
                               (_NW, pad_w))
    src_p = jnp.concatenate([src.reshape(_NW, per_w_real), pad_src], axis=1).reshape(-1)
    dst_p = jnp.concatenate([dst.reshape(_NW, per_w_real), pad_dst], axis=1).reshape(-1)
    extra = (jnp.arange(_CHUNK, dtype=jnp.int32) * 13) % n
    src_p = jnp.concatenate([src_p, extra])
    dst_p = jnp.concatenate([dst_p, jnp.full((_CHUNK,), junk_row, jnp.int32)])
    zero_block = jnp.zeros((agg_rows // _NS, d), jnp.float32)

    seg_sum = _make_segment_sum(n, d, e_pad, agg_rows)

    h = x
    for i in range(num_layers):
        parts = seg_sum(h, src_p, dst_p, zero_block)
        scale = (1.0 + eps[i]).reshape(1)
        dense = _make_dense(n, d, final_relu=(i < num_layers - 1))
        h = dense(scale, h, parts[0], parts[1],
                  W1[i], b1[i].reshape(1, d), g1[i].reshape(1, d),
                  be1[i].reshape(1, d), W2[i], b2[i].reshape(1, d),
                  g2[i].reshape(1, d), be2[i].reshape(1, d),
                  bn_g[i].reshape(1, d), bn_b[i].reshape(1, d))
    return h


# speedup vs baseline: 3.1744x; 1.5063x over previous
import functools

import jax
import jax.numpy as jnp
from jax import lax
from jax.experimental import pallas as pl
from jax.experimental.pallas import tpu as pltpu
from jax.experimental.pallas import tpu_sc as plsc

_NC = 2
_NS = 16
_NW = _NC * _NS
_CHUNK = 128


def _make_segment_sum(n, d, e_pad, agg_rows):
    per_w = e_pad // _NW
    n_chunks = per_w // _CHUNK
    rows_per_tile = agg_rows // _NS
    mesh = plsc.VectorSubcoreMesh(core_axis_name="c", subcore_axis_name="s")

    def body(h_hbm, src_hbm, dst_hbm, zero_hbm, out_hbm,
             agg_sh, src0, src1, dst0, dst1, rows0, rows1, sem0, sem1):
        c = lax.axis_index("c")
        s = lax.axis_index("s")
        wid = s * _NC + c
        src_v = (src0, src1)
        dst_v = (dst0, dst1)
        rows_v = (rows0, rows1)
        sem = (sem0, sem1)
        pltpu.sync_copy(zero_hbm, agg_sh.at[pl.ds(s * rows_per_tile, rows_per_tile)])
        plsc.subcore_barrier()

        base = wid * per_w

        pltpu.sync_copy(src_hbm.at[pl.ds(base, _CHUNK)], src_v[0])
        pltpu.sync_copy(dst_hbm.at[pl.ds(base, _CHUNK)], dst_v[0])
        pltpu.async_copy(h_hbm.at[src_v[0]], rows_v[0], sem[0])

        def pair(g, carry):
            for b in (0, 1):
                k = 2 * g + b
                ob = 1 - b
                off_n = base + (k + 1) * _CHUNK
                pltpu.sync_copy(src_hbm.at[pl.ds(off_n, _CHUNK)], src_v[ob])
                pltpu.sync_copy(dst_hbm.at[pl.ds(off_n, _CHUNK)], dst_v[ob])
                pltpu.async_copy(h_hbm.at[src_v[ob]], rows_v[ob], sem[ob])
                pltpu.make_async_copy(h_hbm.at[src_v[b]], rows_v[b], sem[b]).wait()
                pltpu.sync_copy(rows_v[b], agg_sh.at[dst_v[b]], add=True)
            return carry

        lax.fori_loop(0, n_chunks // 2, pair, 0)
        pltpu.make_async_copy(h_hbm.at[src_v[0]], rows_v[0], sem[0]).wait()
        plsc.subcore_barrier()
        pltpu.sync_copy(agg_sh.at[pl.ds(s * rows_per_tile, rows_per_tile)],
                        out_hbm.at[c, pl.ds(s * rows_per_tile, rows_per_tile)])

    return pl.kernel(
        body,
        out_type=jax.ShapeDtypeStruct((_NC, agg_rows, d), jnp.float32),
        mesh=mesh,
        scratch_types=[
            pltpu.VMEM_SHARED((agg_rows, d), jnp.float32),
            pltpu.VMEM((_CHUNK,), jnp.int32),
            pltpu.VMEM((_CHUNK,), jnp.int32),
            pltpu.VMEM((_CHUNK,), jnp.int32),
            pltpu.VMEM((_CHUNK,), jnp.int32),
            pltpu.VMEM((_CHUNK, d), jnp.float32),
            pltpu.VMEM((_CHUNK, d), jnp.float32),
            pltpu.SemaphoreType.DMA,
            pltpu.SemaphoreType.DMA,
        ],
    )


def _bn(x, g, b):
    mu = jnp.mean(x, axis=0, keepdims=True)
    var = jnp.mean((x - mu) ** 2, axis=0, keepdims=True)
    return g * (x - mu) * lax.rsqrt(var + 1e-5) + b


def _dense_body(scale_ref, h_ref, p0_ref, p1_ref, w1_ref, b1_ref, g1_ref,
                be1_ref, w2_ref, b2_ref, g2_ref, be2_ref, bng_ref, bnb_ref,
                out_ref, *, n, final_relu):
    agg = p0_ref[0:n, :] + p1_ref[0:n, :]
    h2 = h_ref[...] * scale_ref[0] + agg
    a = jnp.dot(h2, w1_ref[...], preferred_element_type=jnp.float32) + b1_ref[...]
    a = jnp.maximum(_bn(a, g1_ref[...], be1_ref[...]), 0.0)
    a = jnp.dot(a, w2_ref[...], preferred_element_type=jnp.float32) + b2_ref[...]
    a = jnp.maximum(_bn(a, g2_ref[...], be2_ref[...]), 0.0)
    a = _bn(a, bng_ref[...], bnb_ref[...])
    if final_relu:
        a = jnp.maximum(a, 0.0)
    out_ref[...] = a


def _make_dense(n, d, final_relu):
    vmem = pl.BlockSpec(memory_space=pltpu.VMEM)
    return pl.pallas_call(
        functools.partial(_dense_body, n=n, final_relu=final_relu),
        out_shape=jax.ShapeDtypeStruct((n, d), jnp.float32),
        in_specs=[pl.BlockSpec(memory_space=pltpu.SMEM)] + [vmem] * 13,
        out_specs=vmem,
    )


def kernel(x, edge_index, eps, W1, b1, g1, be1, W2, b2, g2, be2, bn_g, bn_b):
    n, d = x.shape
    e = edge_index.shape[1]
    num_layers = W1.shape[0]

    agg_rows = ((n + _NS - 1) // _NS + 7) // 8 * 8 * _NS
    junk_row = n
    quantum = 2 * _NW * _CHUNK
    e_pad = ((e + quantum - 1) // quantum) * quantum

    src = edge_index[0]
    dst = edge_index[1]
    per_w_real = e // _NW
    per_w = e_pad // _NW
    pad_w = per_w - per_w_real
    n_junk = agg_rows - n
    pad_src = jnp.broadcast_to((jnp.arange(pad_w, dtype=jnp.int32) * 41) % n,
                               (_NW, pad_w))
    pad_dst = jnp.broadcast_to(junk_row + (jnp.arange(pad_w, dtype=jnp.int32) % n_junk),
                               (_NW, pad_w))
    src_p = jnp.concatenate([src.reshape(_NW, per_w_real), pad_src], axis=1).reshape(-1)
    dst_p = jnp.concatenate([dst.reshape(_NW, per_w_real), pad_dst], axis=1).reshape(-1)
    extra = (jnp.arange(_CHUNK, dtype=jnp.int32) * 13) % n
    src_p = jnp.concatenate([src_p, extra])
    dst_p = jnp.concatenate([dst_p, jnp.full((_CHUNK,), junk_row, jnp.int32)])
    zero_block = jnp.zeros((agg_rows // _NS, d), jnp.float32)

    seg_sum = _make_segment_sum(n, d, e_pad, agg_rows)

    h = x
    for i in range(num_layers):
        parts = seg_sum(h, src_p, dst_p, zero_block)
        scale = (1.0 + eps[i]).reshape(1)
        dense = _make_dense(n, d, final_relu=(i < num_layers - 1))
        h = dense(scale, h, parts[0], parts[1],
                  W1[i], b1[i].reshape(1, d), g1[i].reshape(1, d),
                  be1[i].reshape(1, d), W2[i], b2[i].reshape(1, d),
                  g2[i].reshape(1, d), be2[i].reshape(1, d),
                  bn_g[i].reshape(1, d), bn_b[i].reshape(1, d))
    return h


# speedup vs baseline: 3.6919x; 1.1630x over previous
import functools

import jax
import jax.numpy as jnp
from jax import lax
from jax.experimental import pallas as pl
from jax.experimental.pallas import tpu as pltpu
from jax.experimental.pallas import tpu_sc as plsc

_NC = 2
_NS = 16
_NW = _NC * _NS
_CHUNK = 128


def _make_segment_sum(n, d, e_pad, agg_rows):
    per_w = e_pad // _NW
    n_chunks = per_w // _CHUNK
    rows_per_tile = agg_rows // _NS
    mesh = plsc.VectorSubcoreMesh(core_axis_name="c", subcore_axis_name="s")

    def body(h_hbm, src_hbm, dst_hbm, zero_hbm, out_hbm, agg_sh,
             src0, src1, dst0, dst1, rows0, rows1, sg0, sg1,
             sis0, sis1, sid0, sid1):
        c = lax.axis_index("c")
        s = lax.axis_index("s")
        wid = s * _NC + c
        src_v = (src0, src1)
        dst_v = (dst0, dst1)
        rows_v = (rows0, rows1)
        sg = (sg0, sg1)
        sis = (sis0, sis1)
        sid = (sid0, sid1)
        base = wid * per_w

        def issue_idx(off, b):
            pltpu.async_copy(src_hbm.at[pl.ds(off, _CHUNK)], src_v[b], sis[b])
            pltpu.async_copy(dst_hbm.at[pl.ds(off, _CHUNK)], dst_v[b], sid[b])

        def wait_idx(b):
            pltpu.make_async_copy(src_hbm.at[pl.ds(base, _CHUNK)], src_v[b], sis[b]).wait()
            pltpu.make_async_copy(dst_hbm.at[pl.ds(base, _CHUNK)], dst_v[b], sid[b]).wait()

        def wait_gather(b):
            pltpu.make_async_copy(h_hbm.at[src_v[b]], rows_v[b], sg[b]).wait()

        pltpu.sync_copy(zero_hbm, agg_sh.at[pl.ds(s * rows_per_tile, rows_per_tile)])
        issue_idx(base, 0)
        issue_idx(base + _CHUNK, 1)
        wait_idx(0)
        pltpu.async_copy(h_hbm.at[src_v[0]], rows_v[0], sg[0])
        plsc.subcore_barrier()

        def pair(g, carry):
            for b in (0, 1):
                k = 2 * g + b
                ob = 1 - b
                wait_idx(ob)
                pltpu.async_copy(h_hbm.at[src_v[ob]], rows_v[ob], sg[ob])
                wait_gather(b)
                pltpu.sync_copy(rows_v[b], agg_sh.at[dst_v[b]], add=True)
                issue_idx(base + lax.rem(k + 2, n_chunks) * _CHUNK, b)
            return carry

        lax.fori_loop(0, n_chunks // 2, pair, 0)
        wait_gather(0)
        wait_idx(1)
        plsc.subcore_barrier()
        pltpu.sync_copy(agg_sh.at[pl.ds(s * rows_per_tile, rows_per_tile)],
                        out_hbm.at[c, pl.ds(s * rows_per_tile, rows_per_tile)])

    return pl.kernel(
        body,
        out_type=jax.ShapeDtypeStruct((_NC, agg_rows, d), jnp.float32),
        mesh=mesh,
        scratch_types=[
            pltpu.VMEM_SHARED((agg_rows, d), jnp.float32),
            pltpu.VMEM((_CHUNK,), jnp.int32),
            pltpu.VMEM((_CHUNK,), jnp.int32),
            pltpu.VMEM((_CHUNK,), jnp.int32),
            pltpu.VMEM((_CHUNK,), jnp.int32),
            pltpu.VMEM((_CHUNK, d), jnp.float32),
            pltpu.VMEM((_CHUNK, d), jnp.float32),
            pltpu.SemaphoreType.DMA,
            pltpu.SemaphoreType.DMA,
            pltpu.SemaphoreType.DMA,
            pltpu.SemaphoreType.DMA,
            pltpu.SemaphoreType.DMA,
            pltpu.SemaphoreType.DMA,
        ],
    )


def _bn(x, g, b):
    mu = jnp.mean(x, axis=0, keepdims=True)
    var = jnp.mean((x - mu) ** 2, axis=0, keepdims=True)
    return g * (x - mu) * lax.rsqrt(var + 1e-5) + b


def _dense_body(scale_ref, h_ref, p0_ref, p1_ref, w1_ref, b1_ref, g1_ref,
                be1_ref, w2_ref, b2_ref, g2_ref, be2_ref, bng_ref, bnb_ref,
                out_ref, *, n, final_relu):
    agg = p0_ref[0:n, :] + p1_ref[0:n, :]
    h2 = h_ref[...] * scale_ref[0] + agg
    a = jnp.dot(h2, w1_ref[...], preferred_element_type=jnp.float32) + b1_ref[...]
    a = jnp.maximum(_bn(a, g1_ref[...], be1_ref[...]), 0.0)
    a = jnp.dot(a, w2_ref[...], preferred_element_type=jnp.float32) + b2_ref[...]
    a = jnp.maximum(_bn(a, g2_ref[...], be2_ref[...]), 0.0)
    a = _bn(a, bng_ref[...], bnb_ref[...])
    if final_relu:
        a = jnp.maximum(a, 0.0)
    out_ref[...] = a


def _make_dense(n, d, final_relu):
    vmem = pl.BlockSpec(memory_space=pltpu.VMEM)
    return pl.pallas_call(
        functools.partial(_dense_body, n=n, final_relu=final_relu),
        out_shape=jax.ShapeDtypeStruct((n, d), jnp.float32),
        in_specs=[pl.BlockSpec(memory_space=pltpu.SMEM)] + [vmem] * 13,
        out_specs=vmem,
    )


def kernel(x, edge_index, eps, W1, b1, g1, be1, W2, b2, g2, be2, bn_g, bn_b):
    n, d = x.shape
    e = edge_index.shape[1]
    num_layers = W1.shape[0]

    agg_rows = ((n + _NS - 1) // _NS + 7) // 8 * 8 * _NS
    junk_row = n
    quantum = 2 * _NW * _CHUNK
    e_pad = ((e + quantum - 1) // quantum) * quantum

    src = edge_index[0]
    dst = edge_index[1]
    per_w_real = e // _NW
    per_w = e_pad // _NW
    pad_w = per_w - per_w_real
    n_junk = agg_rows - n
    pad_src = jnp.broadcast_to((jnp.arange(pad_w, dtype=jnp.int32) * 41) % n,
                               (_NW, pad_w))
    pad_dst = jnp.broadcast_to(junk_row + (jnp.arange(pad_w, dtype=jnp.int32) % n_junk),
                               (_NW, pad_w))
    src_p = jnp.concatenate([src.reshape(_NW, per_w_real), pad_src], axis=1).reshape(-1)
    dst_p = jnp.concatenate([dst.reshape(_NW, per_w_real), pad_dst], axis=1).reshape(-1)
    zero_block = jnp.zeros((agg_rows // _NS, d), jnp.float32)

    seg_sum = _make_segment_sum(n, d, e_pad, agg_rows)

    h = x
    for i in range(num_layers):
        parts = seg_sum(h, src_p, dst_p, zero_block)
        scale = (1.0 + eps[i]).reshape(1)
        dense = _make_dense(n, d, final_relu=(i < num_layers - 1))
        h = dense(scale, h, parts[0], parts[1],
                  W1[i], b1[i].reshape(1, d), g1[i].reshape(1, d),
                  be1[i].reshape(1, d), W2[i], b2[i].reshape(1, d),
                  g2[i].reshape(1, d), be2[i].reshape(1, d),
                  bn_g[i].reshape(1, d), bn_b[i].reshape(1, d))
    return h
